# baseline (speedup 1.0000x reference)
import functools

import jax
import jax.numpy as jnp
from jax import lax
from jax.experimental import pallas as pl
from jax.experimental.pallas import tpu as pltpu
from jax.experimental.pallas import tpu_sc as plsc

NX, NY, NZ = 512, 512, 1
F = 64
P = 100000
B = 4
TOTAL = NZ * NY * NX
NCELL = B * TOTAL
NW = 32
PPW = 3136
KP = NW * PPW
CPW = NCELL // NW
KCHUNK = 2048
NKCH = KP // KCHUNK
FP = 128
SEG = NX
NSEG = CPW // SEG
WAVE = 64


@functools.partial(
    pl.kernel,
    mesh=plsc.VectorSubcoreMesh(core_axis_name="c", subcore_axis_name="s"),
    compiler_params=pltpu.CompilerParams(needs_layout_passes=False),
    out_type=jax.ShapeDtypeStruct((KP,), jnp.int32),
    scratch_types=[
        pltpu.VMEM((PPW,), jnp.int32),
        pltpu.VMEM((PPW,), jnp.int32),
        pltpu.VMEM((PPW,), jnp.int32),
        pltpu.VMEM((PPW,), jnp.int32),
    ],
)
def _keys_kernel(bc_hbm, yc_hbm, xc_hbm, keys_hbm, bbuf, ybuf, xbuf, kbuf):
    w = lax.axis_index("s") * 2 + lax.axis_index("c")
    base = w * PPW
    pltpu.sync_copy(bc_hbm.at[pl.ds(base, PPW)], bbuf)
    pltpu.sync_copy(yc_hbm.at[pl.ds(base, PPW)], ybuf)
    pltpu.sync_copy(xc_hbm.at[pl.ds(base, PPW)], xbuf)
    iota16 = lax.iota(jnp.int32, 16)

    def body(i, carry):
        bv = bbuf[pl.ds(i * 16, 16)]
        yv = ybuf[pl.ds(i * 16, 16)]
        xv = xbuf[pl.ds(i * 16, 16)]
        key = bv * TOTAL + (yv * NX + xv)
        key = jnp.where(base + i * 16 + iota16 < P, key, -1)
        kbuf[pl.ds(i * 16, 16)] = key
        return carry

    lax.fori_loop(0, PPW // 16, body, 0)
    pltpu.sync_copy(kbuf, keys_hbm.at[pl.ds(base, PPW)])


@functools.partial(
    pl.kernel,
    mesh=plsc.VectorSubcoreMesh(core_axis_name="c", subcore_axis_name="s"),
    compiler_params=pltpu.CompilerParams(needs_layout_passes=False),
    out_type=jax.ShapeDtypeStruct((B, F, NY, NX), jnp.float32),
    scratch_types=[
        pltpu.VMEM((CPW,), jnp.int32),
        pltpu.VMEM((2, KCHUNK), jnp.int32),
        pltpu.VMEM((SEG + 64,), jnp.int32),
        pltpu.VMEM((2, SEG), jnp.int32),
        pltpu.VMEM((2, WAVE, FP), jnp.float32),
        pltpu.VMEM((2, F * SEG), jnp.float32),
        pltpu.SemaphoreType.DMA,
        pltpu.SemaphoreType.DMA,
        pltpu.SemaphoreType.DMA,
    ],
)
def _scatter_kernel(keys_hbm, pf_hbm, out_hbm,
                    wid_l, keybuf, gidx, cpos, rows, stage,
                    sem_k, sem_g, sem_o):
    w = lax.axis_index("s") * 2 + lax.axis_index("c")
    iota16 = lax.iota(jnp.int32, 16)
    neg116 = jnp.full((16,), -1, jnp.int32)
    zero16f = jnp.zeros((16,), jnp.float32)

    def init_body(i, c):
        wid_l[pl.ds(i * 16, 16)] = neg116
        return c

    lax.fori_loop(0, CPW // 16, init_body, 0)

    NSV = (F * SEG) // 16

    def zinit_body(i, c):
        stage[i // NSV, pl.ds((i % NSV) * 16, 16)] = zero16f
        return c

    lax.fori_loop(0, 2 * NSV, zinit_body, 0)

    lo = w * CPW
    pltpu.async_copy(keys_hbm.at[pl.ds(0, KCHUNK)], keybuf.at[0], sem_k)

    def chunk_body(c, carry):
        par = c % 2
        pltpu.make_async_copy(keys_hbm.at[pl.ds(c * KCHUNK, KCHUNK)],
                              keybuf.at[par], sem_k).wait()

        @pl.when(c + 1 < NKCH)
        def _():
            pltpu.async_copy(keys_hbm.at[pl.ds((c + 1) * KCHUNK, KCHUNK)],
                             keybuf.at[(c + 1) % 2], sem_k)

        def vec_body(i, cc):
            k16 = keybuf[par, pl.ds(i * 16, 16)]
            off = k16 - lo
            m = off.astype(jnp.uint32) < jnp.uint32(CPW)

            @pl.when(jnp.any(m))
            def _():
                offs = jnp.where(m, off, 0)
                pvec = c * KCHUNK + i * 16 + iota16
                plsc.store_scatter(wid_l, [offs], pvec, mask=m)
                cur = plsc.load_gather(wid_l, [offs], mask=m)
                need = m & (pvec > cur)

                def w_cond(st):
                    return st[0]

                def w_body(st):
                    nd = st[1]
                    plsc.store_scatter(wid_l, [offs], pvec, mask=nd)
                    cur2 = plsc.load_gather(wid_l, [offs], mask=m)
                    nd2 = m & (pvec > cur2)
                    return jnp.any(nd2), nd2

                lax.while_loop(w_cond, w_body, (jnp.any(need), need))

            return cc

        lax.fori_loop(0, KCHUNK // 16, vec_body, 0)
        return carry

    lax.fori_loop(0, NKCH, chunk_body, 0)

    b_w = w // 8
    y0 = (w % 8) * NSEG

    def piece_body(s, cnts):
        cnt_m2, cnt_m1 = cnts
        spar = s % 2
        parv = jnp.zeros((16,), jnp.int32) + spar

        @pl.when(s >= 2)
        def _():
            for f in range(F):
                pltpu.make_async_copy(stage.at[spar, pl.ds(f * SEG, SEG)],
                                      out_hbm.at[b_w, f, y0 + s, :],
                                      sem_o).wait()

            def rz_body(j, c):
                valid = (j * 16 + iota16) < cnt_m2
                cp16 = jnp.where(valid, cpos[spar, pl.ds(j * 16, 16)], 0)
                for f in range(F):
                    plsc.store_scatter(stage, [parv, cp16 + f * SEG],
                                       zero16f, mask=valid)
                return c

            lax.fori_loop(0, (cnt_m2 + 15) // 16, rz_body, 0)

        def comp_body(i, cnt):
            w16 = wid_l[pl.ds(s * SEG + i * 16, 16)]
            m = w16 >= 0
            mi = m.astype(jnp.int32)
            pos = cnt + plsc.cumsum(mi) - 1
            poss = jnp.where(m, pos, 0)
            plsc.store_scatter(gidx, [poss], w16, mask=m)
            plsc.store_scatter(cpos, [parv, poss], i * 16 + iota16, mask=m)
            return cnt + jnp.sum(mi)

        cnt = lax.fori_loop(0, SEG // 16, comp_body, jnp.int32(0))

        for t in range(WAVE // 16):
            plsc.store_scatter(gidx, [cnt + t * 16 + iota16], t * 16 + iota16)

        ntr = (cnt + WAVE - 1) // WAVE

        @pl.when(cnt > 0)
        def _():
            pltpu.async_copy(pf_hbm.at[gidx.at[pl.ds(0, WAVE)]],
                             rows.at[0], sem_g)

        def wave_body(t, c):
            tpar = t % 2
            pltpu.make_async_copy(pf_hbm.at[gidx.at[pl.ds(t * WAVE, WAVE)]],
                                  rows.at[tpar], sem_g).wait()

            @pl.when(t + 1 < ntr)
            def _():
                pltpu.async_copy(
                    pf_hbm.at[gidx.at[pl.ds((t + 1) * WAVE, WAVE)]],
                    rows.at[(t + 1) % 2], sem_g)

            tparv = jnp.zeros((16,), jnp.int32) + tpar
            lc = cnt - t * WAVE
            for q in range(WAVE // 16):
                valid = (q * 16 + iota16) < lc
                cp16 = jnp.where(
                    valid, cpos[spar, pl.ds(t * WAVE + q * 16, 16)], 0)
                r16 = q * 16 + iota16
                for f in range(F):
                    fv = jnp.full((16,), f, jnp.int32)
                    vals = plsc.load_gather(rows, [tparv, r16, fv])
                    plsc.store_scatter(stage, [parv, cp16 + f * SEG], vals,
                                       mask=valid)
            return c

        lax.fori_loop(0, ntr, wave_body, 0)

        for f in range(F):
            pltpu.async_copy(stage.at[spar, pl.ds(f * SEG, SEG)],
                             out_hbm.at[b_w, f, y0 + s, :], sem_o)
        return (cnt_m1, cnt)

    lax.fori_loop(0, NSEG, piece_body, (jnp.int32(0), jnp.int32(0)))

    def drain_body(i, c):
        pltpu.make_async_copy(stage.at[0, pl.ds(0, SEG)],
                              out_hbm.at[0, 0, 0, :], sem_o).wait()
        return c

    lax.fori_loop(0, 2 * F, drain_body, 0)


def kernel(pillar_features, voxel_coords, voxel_features):
    del voxel_features
    vc = voxel_coords.astype(jnp.int32)
    bcol = jnp.zeros((KP,), jnp.int32).at[:P].set(vc[:, 0])
    ycol = jnp.zeros((KP,), jnp.int32).at[:P].set(vc[:, 2])
    xcol = jnp.zeros((KP,), jnp.int32).at[:P].set(vc[:, 3])
    pf_pad = jnp.zeros((P + 16, FP), jnp.float32).at[:P, :F].set(pillar_features)
    keys = _keys_kernel(bcol, ycol, xcol)
    out = _scatter_kernel(keys, pf_pad)
    return out.reshape(B, F * NZ, NY, NX)


# speedup vs baseline: 1.1363x; 1.1363x over previous
import functools

import jax
import jax.numpy as jnp
from jax import lax
from jax.experimental import pallas as pl
from jax.experimental.pallas import tpu as pltpu
from jax.experimental.pallas import tpu_sc as plsc

NX, NY, NZ = 512, 512, 1
F = 64
P = 100000
B = 4
TOTAL = NZ * NY * NX
NCELL = B * TOTAL
NW = 32
PPW = 3136
KP = NW * PPW
CPW = NCELL // NW
KCHUNK = 2048
NKCH = KP // KCHUNK
FP = 128
SEG = NX
NSEG = CPW // SEG
WAVE = 64


@functools.partial(
    pl.kernel,
    mesh=plsc.VectorSubcoreMesh(core_axis_name="c", subcore_axis_name="s"),
    compiler_params=pltpu.CompilerParams(needs_layout_passes=False),
    out_type=jax.ShapeDtypeStruct((KP,), jnp.int32),
    scratch_types=[
        pltpu.VMEM((PPW,), jnp.int32),
        pltpu.VMEM((PPW,), jnp.int32),
        pltpu.VMEM((PPW,), jnp.int32),
        pltpu.VMEM((PPW,), jnp.int32),
    ],
)
def _keys_kernel(bc_hbm, yc_hbm, xc_hbm, keys_hbm, bbuf, ybuf, xbuf, kbuf):
    w = lax.axis_index("s") * 2 + lax.axis_index("c")
    base = w * PPW
    pltpu.sync_copy(bc_hbm.at[pl.ds(base, PPW)], bbuf)
    pltpu.sync_copy(yc_hbm.at[pl.ds(base, PPW)], ybuf)
    pltpu.sync_copy(xc_hbm.at[pl.ds(base, PPW)], xbuf)
    iota16 = lax.iota(jnp.int32, 16)

    def body(i, carry):
        bv = bbuf[pl.ds(i * 16, 16)]
        yv = ybuf[pl.ds(i * 16, 16)]
        xv = xbuf[pl.ds(i * 16, 16)]
        key = bv * TOTAL + (yv * NX + xv)
        key = jnp.where(base + i * 16 + iota16 < P, key, -1)
        kbuf[pl.ds(i * 16, 16)] = key
        return carry

    lax.fori_loop(0, PPW // 16, body, 0)
    pltpu.sync_copy(kbuf, keys_hbm.at[pl.ds(base, PPW)])


@functools.partial(
    pl.kernel,
    mesh=plsc.VectorSubcoreMesh(core_axis_name="c", subcore_axis_name="s"),
    compiler_params=pltpu.CompilerParams(needs_layout_passes=False),
    out_type=jax.ShapeDtypeStruct((B, F, NY, NX), jnp.float32),
    scratch_types=[
        pltpu.VMEM((CPW,), jnp.int32),
        pltpu.VMEM((2, KCHUNK), jnp.int32),
        pltpu.VMEM((SEG + 64,), jnp.int32),
        pltpu.VMEM((2, SEG), jnp.int32),
        pltpu.VMEM((2, WAVE, FP), jnp.float32),
        pltpu.VMEM((2, F * SEG), jnp.float32),
        pltpu.SemaphoreType.DMA,
        pltpu.SemaphoreType.DMA,
        pltpu.SemaphoreType.DMA,
    ],
)
def _scatter_kernel(keys_hbm, pf_hbm, out_hbm,
                    wid_l, keybuf, gidx, cpos, rows, stage,
                    sem_k, sem_g, sem_o):
    w = lax.axis_index("s") * 2 + lax.axis_index("c")
    iota16 = lax.iota(jnp.int32, 16)
    neg116 = jnp.full((16,), -1, jnp.int32)
    zero16f = jnp.zeros((16,), jnp.float32)

    def init_body(i, c):
        wid_l[pl.ds(i * 16, 16)] = neg116
        return c

    lax.fori_loop(0, CPW // 16, init_body, 0)

    NSV = (F * SEG) // 16

    def zinit_body(i, c):
        stage[i // NSV, pl.ds((i % NSV) * 16, 16)] = zero16f
        return c

    lax.fori_loop(0, 2 * NSV, zinit_body, 0)

    lo = w * CPW
    pltpu.async_copy(keys_hbm.at[pl.ds(0, KCHUNK)], keybuf.at[0], sem_k)

    def chunk_body(c, carry):
        par = c % 2
        pltpu.make_async_copy(keys_hbm.at[pl.ds(c * KCHUNK, KCHUNK)],
                              keybuf.at[par], sem_k).wait()

        @pl.when(c + 1 < NKCH)
        def _():
            pltpu.async_copy(keys_hbm.at[pl.ds((c + 1) * KCHUNK, KCHUNK)],
                             keybuf.at[(c + 1) % 2], sem_k)

        def vec_body(i, cc):
            k16 = keybuf[par, pl.ds(i * 16, 16)]
            off = k16 - lo
            m = off.astype(jnp.uint32) < jnp.uint32(CPW)
            offs = jnp.where(m, off, 0)
            pvec = c * KCHUNK + i * 16 + iota16
            plsc.store_scatter(wid_l, [offs], pvec, mask=m)
            cur = plsc.load_gather(wid_l, [offs], mask=m)
            need = m & (pvec > cur)

            def w_cond(st):
                return st[0]

            def w_body(st):
                nd = st[1]
                plsc.store_scatter(wid_l, [offs], pvec, mask=nd)
                cur2 = plsc.load_gather(wid_l, [offs], mask=m)
                nd2 = m & (pvec > cur2)
                return jnp.any(nd2), nd2

            lax.while_loop(w_cond, w_body, (jnp.any(need), need))
            return cc

        lax.fori_loop(0, KCHUNK // 16, vec_body, 0)
        return carry

    lax.fori_loop(0, NKCH, chunk_body, 0)

    b_w = w // 8
    y0 = (w % 8) * NSEG

    def piece_body(s, cnts):
        cnt_m2, cnt_m1 = cnts
        spar = s % 2
        parv = jnp.zeros((16,), jnp.int32) + spar

        @pl.when(s >= 2)
        def _():
            for f in range(F):
                pltpu.make_async_copy(stage.at[spar, pl.ds(f * SEG, SEG)],
                                      out_hbm.at[b_w, f, y0 + s, :],
                                      sem_o).wait()

            def rz_body(j, c):
                valid = (j * 16 + iota16) < cnt_m2
                cp16 = jnp.where(valid, cpos[spar, pl.ds(j * 16, 16)], 0)
                for f in range(F):
                    plsc.store_scatter(stage, [parv, cp16 + f * SEG],
                                       zero16f, mask=valid)
                return c

            lax.fori_loop(0, (cnt_m2 + 15) // 16, rz_body, 0)

        def comp_body(i, cnt):
            w16 = wid_l[pl.ds(s * SEG + i * 16, 16)]
            m = w16 >= 0
            mi = m.astype(jnp.int32)
            pos = cnt + plsc.cumsum(mi) - 1
            poss = jnp.where(m, pos, 0)
            plsc.store_scatter(gidx, [poss], w16, mask=m)
            plsc.store_scatter(cpos, [parv, poss], i * 16 + iota16, mask=m)
            return cnt + jnp.sum(mi)

        cnt = lax.fori_loop(0, SEG // 16, comp_body, jnp.int32(0))

        for t in range(WAVE // 16):
            plsc.store_scatter(gidx, [cnt + t * 16 + iota16], t * 16 + iota16)

        ntr = (cnt + WAVE - 1) // WAVE

        @pl.when(cnt > 0)
        def _():
            pltpu.async_copy(pf_hbm.at[gidx.at[pl.ds(0, WAVE)]],
                             rows.at[0], sem_g)

        def wave_body(t, c):
            tpar = t % 2
            pltpu.make_async_copy(pf_hbm.at[gidx.at[pl.ds(t * WAVE, WAVE)]],
                                  rows.at[tpar], sem_g).wait()

            @pl.when(t + 1 < ntr)
            def _():
                pltpu.async_copy(
                    pf_hbm.at[gidx.at[pl.ds((t + 1) * WAVE, WAVE)]],
                    rows.at[(t + 1) % 2], sem_g)

            tparv = jnp.zeros((16,), jnp.int32) + tpar
            lc = cnt - t * WAVE
            for q in range(WAVE // 16):
                valid = (q * 16 + iota16) < lc
                cp16 = jnp.where(
                    valid, cpos[spar, pl.ds(t * WAVE + q * 16, 16)], 0)
                r16 = q * 16 + iota16
                for f in range(F):
                    fv = jnp.full((16,), f, jnp.int32)
                    vals = plsc.load_gather(rows, [tparv, r16, fv])
                    plsc.store_scatter(stage, [parv, cp16 + f * SEG], vals,
                                       mask=valid)
            return c

        lax.fori_loop(0, ntr, wave_body, 0)

        for f in range(F):
            pltpu.async_copy(stage.at[spar, pl.ds(f * SEG, SEG)],
                             out_hbm.at[b_w, f, y0 + s, :], sem_o)
        return (cnt_m1, cnt)

    lax.fori_loop(0, NSEG, piece_body, (jnp.int32(0), jnp.int32(0)))

    def drain_body(i, c):
        pltpu.make_async_copy(stage.at[0, pl.ds(0, SEG)],
                              out_hbm.at[0, 0, 0, :], sem_o).wait()
        return c

    lax.fori_loop(0, 2 * F, drain_body, 0)


def kernel(pillar_features, voxel_coords, voxel_features):
    del voxel_features
    vc = voxel_coords.astype(jnp.int32)
    bcol = jnp.zeros((KP,), jnp.int32).at[:P].set(vc[:, 0])
    ycol = jnp.zeros((KP,), jnp.int32).at[:P].set(vc[:, 2])
    xcol = jnp.zeros((KP,), jnp.int32).at[:P].set(vc[:, 3])
    pf_pad = jnp.zeros((P + 16, FP), jnp.float32).at[:P, :F].set(pillar_features)
    keys = _keys_kernel(bcol, ycol, xcol)
    out = _scatter_kernel(keys, pf_pad)
    return out.reshape(B, F * NZ, NY, NX)


# speedup vs baseline: 1.6778x; 1.4766x over previous
import functools

import jax
import jax.numpy as jnp
from jax import lax
from jax.experimental import pallas as pl
from jax.experimental.pallas import tpu as pltpu
from jax.experimental.pallas import tpu_sc as plsc

NX, NY, NZ = 512, 512, 1
F = 64
P = 100000
B = 4
TOTAL = NZ * NY * NX
NCELL = B * TOTAL
NW = 32
PPW = 3136
KP = NW * PPW
CPW = NCELL // NW
KCHUNK = 2048
NKCH = KP // KCHUNK
FP = 128
SEG = NX
NSEG = CPW // SEG
WAVE = 64


@functools.partial(
    pl.kernel,
    mesh=plsc.VectorSubcoreMesh(core_axis_name="c", subcore_axis_name="s"),
    compiler_params=pltpu.CompilerParams(needs_layout_passes=False),
    out_type=jax.ShapeDtypeStruct((KP,), jnp.int32),
    scratch_types=[
        pltpu.VMEM((PPW,), jnp.int32),
        pltpu.VMEM((PPW,), jnp.int32),
        pltpu.VMEM((PPW,), jnp.int32),
        pltpu.VMEM((PPW,), jnp.int32),
    ],
)
def _keys_kernel(bc_hbm, yc_hbm, xc_hbm, keys_hbm, bbuf, ybuf, xbuf, kbuf):
    w = lax.axis_index("s") * 2 + lax.axis_index("c")
    base = w * PPW
    pltpu.sync_copy(bc_hbm.at[pl.ds(base, PPW)], bbuf)
    pltpu.sync_copy(yc_hbm.at[pl.ds(base, PPW)], ybuf)
    pltpu.sync_copy(xc_hbm.at[pl.ds(base, PPW)], xbuf)
    iota16 = lax.iota(jnp.int32, 16)

    def body(i, carry):
        bv = bbuf[pl.ds(i * 16, 16)]
        yv = ybuf[pl.ds(i * 16, 16)]
        xv = xbuf[pl.ds(i * 16, 16)]
        key = bv * TOTAL + (yv * NX + xv)
        key = jnp.where(base + i * 16 + iota16 < P, key, -1)
        kbuf[pl.ds(i * 16, 16)] = key
        return carry

    lax.fori_loop(0, PPW // 16, body, 0)
    pltpu.sync_copy(kbuf, keys_hbm.at[pl.ds(base, PPW)])


@functools.partial(
    pl.kernel,
    mesh=plsc.VectorSubcoreMesh(core_axis_name="c", subcore_axis_name="s"),
    compiler_params=pltpu.CompilerParams(needs_layout_passes=False),
    out_type=jax.ShapeDtypeStruct((B, F, NY, NX), jnp.float32),
    scratch_types=[
        pltpu.VMEM((CPW,), jnp.int32),
        pltpu.VMEM((2, KCHUNK), jnp.int32),
        pltpu.VMEM((SEG + 64,), jnp.int32),
        pltpu.VMEM((2, SEG), jnp.int32),
        pltpu.VMEM((2, WAVE, FP), jnp.float32),
        pltpu.VMEM((2, F, SEG), jnp.float32),
        pltpu.SemaphoreType.DMA,
        pltpu.SemaphoreType.DMA,
        pltpu.SemaphoreType.DMA,
    ],
)
def _scatter_kernel(keys_hbm, pf_hbm, out_hbm,
                    wid_l, keybuf, gidx, cpos, rows, stage,
                    sem_k, sem_g, sem_o):
    w = lax.axis_index("s") * 2 + lax.axis_index("c")
    iota16 = lax.iota(jnp.int32, 16)
    neg116 = jnp.full((16,), -1, jnp.int32)
    zero16f = jnp.zeros((16,), jnp.float32)

    def init_body(i, c):
        wid_l[pl.ds(i * 16, 16)] = neg116
        return c

    lax.fori_loop(0, CPW // 16, init_body, 0)

    for zpar in range(2):
        def zinit_body(i, c, zpar=zpar):
            stage[zpar, i // (SEG // 16), pl.ds((i % (SEG // 16)) * 16, 16)] = zero16f
            return c

        lax.fori_loop(0, (F * SEG) // 16, zinit_body, 0)

    lo = w * CPW
    pltpu.async_copy(keys_hbm.at[pl.ds(0, KCHUNK)], keybuf.at[0], sem_k)

    def chunk_body(c, carry):
        par = c % 2
        pltpu.make_async_copy(keys_hbm.at[pl.ds(c * KCHUNK, KCHUNK)],
                              keybuf.at[par], sem_k).wait()

        @pl.when(c + 1 < NKCH)
        def _():
            pltpu.async_copy(keys_hbm.at[pl.ds((c + 1) * KCHUNK, KCHUNK)],
                             keybuf.at[(c + 1) % 2], sem_k)

        def vec_body(i4, acc):
            for u in range(4):
                i = i4 * 4 + u
                k16 = keybuf[par, pl.ds(i * 16, 16)]
                off = k16 - lo
                m = off.astype(jnp.uint32) < jnp.uint32(CPW)
                offs = jnp.where(m, off, 0)
                pvec = c * KCHUNK + i * 16 + iota16
                plsc.store_scatter(wid_l, [offs], pvec, mask=m)
                cur = plsc.load_gather(wid_l, [offs], mask=m)
                acc = acc | (m & (pvec > cur))
            return acc

        acc = lax.fori_loop(0, KCHUNK // 64, vec_body,
                            jnp.zeros((16,), jnp.bool_))

        @pl.when(jnp.any(acc))
        def _():
            def fix_vec(i, cc):
                k16 = keybuf[par, pl.ds(i * 16, 16)]
                off = k16 - lo
                m = off.astype(jnp.uint32) < jnp.uint32(CPW)
                offs = jnp.where(m, off, 0)
                pvec = c * KCHUNK + i * 16 + iota16
                cur = plsc.load_gather(wid_l, [offs], mask=m)
                need = m & (pvec > cur)

                def w_cond(st):
                    return st[0]

                def w_body(st):
                    nd = st[1]
                    plsc.store_scatter(wid_l, [offs], pvec, mask=nd)
                    cur2 = plsc.load_gather(wid_l, [offs], mask=m)
                    nd2 = m & (pvec > cur2)
                    return jnp.any(nd2), nd2

                lax.while_loop(w_cond, w_body, (jnp.any(need), need))
                return cc

            lax.fori_loop(0, KCHUNK // 16, fix_vec, 0)

        return carry

    lax.fori_loop(0, NKCH, chunk_body, 0)

    b_w = w // 8
    y0 = (w % 8) * NSEG

    def piece_body(s, cnts):
        cnt_m2, cnt_m1 = cnts
        spar = s % 2
        parv = jnp.zeros((16,), jnp.int32) + spar

        @pl.when(s >= 2)
        def _():
            pltpu.make_async_copy(stage.at[spar],
                                  out_hbm.at[b_w, :, y0 + s, :],
                                  sem_o).wait()

            def rz_body(j, c):
                valid = (j * 16 + iota16) < cnt_m2
                cp16 = jnp.where(valid, cpos[spar, pl.ds(j * 16, 16)], 0)
                for f in range(F):
                    fv = jnp.full((16,), f, jnp.int32)
                    plsc.store_scatter(stage, [parv, fv, cp16],
                                       zero16f, mask=valid)
                return c

            lax.fori_loop(0, (cnt_m2 + 15) // 16, rz_body, 0)

        def comp_body(i, cnt):
            w16 = wid_l[pl.ds(s * SEG + i * 16, 16)]
            m = w16 >= 0
            mi = m.astype(jnp.int32)
            pos = cnt + plsc.cumsum(mi) - 1
            poss = jnp.where(m, pos, 0)
            plsc.store_scatter(gidx, [poss], w16, mask=m)
            plsc.store_scatter(cpos, [parv, poss], i * 16 + iota16, mask=m)
            return cnt + jnp.sum(mi)

        cnt = lax.fori_loop(0, SEG // 16, comp_body, jnp.int32(0))

        for t in range(WAVE // 16):
            plsc.store_scatter(gidx, [cnt + t * 16 + iota16], t * 16 + iota16)

        ntr = (cnt + WAVE - 1) // WAVE

        @pl.when(cnt > 0)
        def _():
            pltpu.async_copy(pf_hbm.at[gidx.at[pl.ds(0, WAVE)]],
                             rows.at[0], sem_g)

        def wave_body(t, c):
            tpar = t % 2
            pltpu.make_async_copy(pf_hbm.at[gidx.at[pl.ds(t * WAVE, WAVE)]],
                                  rows.at[tpar], sem_g).wait()

            @pl.when(t + 1 < ntr)
            def _():
                pltpu.async_copy(
                    pf_hbm.at[gidx.at[pl.ds((t + 1) * WAVE, WAVE)]],
                    rows.at[(t + 1) % 2], sem_g)

            tparv = jnp.zeros((16,), jnp.int32) + tpar
            lc = cnt - t * WAVE
            for q in range(WAVE // 16):
                valid = (q * 16 + iota16) < lc
                cp16 = jnp.where(
                    valid, cpos[spar, pl.ds(t * WAVE + q * 16, 16)], 0)
                r16 = q * 16 + iota16
                for f in range(F):
                    fv = jnp.full((16,), f, jnp.int32)
                    vals = plsc.load_gather(rows, [tparv, r16, fv])
                    plsc.store_scatter(stage, [parv, fv, cp16], vals,
                                       mask=valid)
            return c

        lax.fori_loop(0, ntr, wave_body, 0)

        pltpu.async_copy(stage.at[spar], out_hbm.at[b_w, :, y0 + s, :], sem_o)
        return (cnt_m1, cnt)

    lax.fori_loop(0, NSEG, piece_body, (jnp.int32(0), jnp.int32(0)))

    def drain_body(i, c):
        pltpu.make_async_copy(stage.at[0], out_hbm.at[0, :, 0, :],
                              sem_o).wait()
        return c

    lax.fori_loop(0, 2, drain_body, 0)


def kernel(pillar_features, voxel_coords, voxel_features):
    del voxel_features
    vc = voxel_coords.astype(jnp.int32)
    bcol = jnp.zeros((KP,), jnp.int32).at[:P].set(vc[:, 0])
    ycol = jnp.zeros((KP,), jnp.int32).at[:P].set(vc[:, 2])
    xcol = jnp.zeros((KP,), jnp.int32).at[:P].set(vc[:, 3])
    pf_pad = jnp.zeros((P + 16, FP), jnp.float32).at[:P, :F].set(pillar_features)
    keys = _keys_kernel(bcol, ycol, xcol)
    out = _scatter_kernel(keys, pf_pad)
    return out.reshape(B, F * NZ, NY, NX)


# speedup vs baseline: 1.8832x; 1.1225x over previous
import functools

import jax
import jax.numpy as jnp
from jax import lax
from jax.experimental import pallas as pl
from jax.experimental.pallas import tpu as pltpu
from jax.experimental.pallas import tpu_sc as plsc

NX, NY, NZ = 512, 512, 1
F = 64
P = 100000
B = 4
TOTAL = NZ * NY * NX
NCELL = B * TOTAL
NW = 32
PPW = 3136
KP = NW * PPW
CPW = NCELL // NW
KCHUNK = 2048
NKCH = KP // KCHUNK
FP = 128
SEG = NX
NSEG = CPW // SEG
WAVE = 64


@functools.partial(
    pl.kernel,
    mesh=plsc.VectorSubcoreMesh(core_axis_name="c", subcore_axis_name="s"),
    compiler_params=pltpu.CompilerParams(needs_layout_passes=False),
    out_type=jax.ShapeDtypeStruct((KP,), jnp.int32),
    scratch_types=[
        pltpu.VMEM((PPW,), jnp.int32),
        pltpu.VMEM((PPW,), jnp.int32),
        pltpu.VMEM((PPW,), jnp.int32),
        pltpu.VMEM((PPW,), jnp.int32),
    ],
)
def _keys_kernel(bc_hbm, yc_hbm, xc_hbm, keys_hbm, bbuf, ybuf, xbuf, kbuf):
    w = lax.axis_index("s") * 2 + lax.axis_index("c")
    base = w * PPW
    pltpu.sync_copy(bc_hbm.at[pl.ds(base, PPW)], bbuf)
    pltpu.sync_copy(yc_hbm.at[pl.ds(base, PPW)], ybuf)
    pltpu.sync_copy(xc_hbm.at[pl.ds(base, PPW)], xbuf)
    iota16 = lax.iota(jnp.int32, 16)

    def body(i, carry):
        bv = bbuf[pl.ds(i * 16, 16)]
        yv = ybuf[pl.ds(i * 16, 16)]
        xv = xbuf[pl.ds(i * 16, 16)]
        key = bv * TOTAL + (yv * NX + xv)
        key = jnp.where(base + i * 16 + iota16 < P, key, -1)
        kbuf[pl.ds(i * 16, 16)] = key
        return carry

    lax.fori_loop(0, PPW // 16, body, 0)
    pltpu.sync_copy(kbuf, keys_hbm.at[pl.ds(base, PPW)])


@functools.partial(
    pl.kernel,
    mesh=plsc.VectorSubcoreMesh(core_axis_name="c", subcore_axis_name="s"),
    compiler_params=pltpu.CompilerParams(needs_layout_passes=False),
    out_type=jax.ShapeDtypeStruct((B, F, NY, NX), jnp.float32),
    scratch_types=[
        pltpu.VMEM((CPW,), jnp.int32),
        pltpu.VMEM((2, KCHUNK), jnp.int32),
        pltpu.VMEM((2, SEG + 64), jnp.int32),
        pltpu.VMEM((3, SEG), jnp.int32),
        pltpu.VMEM((2, WAVE, FP), jnp.float32),
        pltpu.VMEM((2, F, SEG), jnp.float32),
        pltpu.SemaphoreType.DMA,
        pltpu.SemaphoreType.DMA,
        pltpu.SemaphoreType.DMA,
    ],
)
def _scatter_kernel(keys_hbm, pf_hbm, out_hbm,
                    wid_l, keybuf, gidx, cpos, rows, stage,
                    sem_k, sem_g, sem_o):
    w = lax.axis_index("s") * 2 + lax.axis_index("c")
    iota16 = lax.iota(jnp.int32, 16)
    neg116 = jnp.full((16,), -1, jnp.int32)
    zero16f = jnp.zeros((16,), jnp.float32)

    def init_body(i, c):
        wid_l[pl.ds(i * 16, 16)] = neg116
        return c

    lax.fori_loop(0, CPW // 16, init_body, 0)

    for zpar in range(2):
        def zinit_body(i, c, zpar=zpar):
            stage[zpar, i // (SEG // 16), pl.ds((i % (SEG // 16)) * 16, 16)] = zero16f
            return c

        lax.fori_loop(0, (F * SEG) // 16, zinit_body, 0)

    lo = w * CPW
    pltpu.async_copy(keys_hbm.at[pl.ds(0, KCHUNK)], keybuf.at[0], sem_k)

    def chunk_body(c, carry):
        par = c % 2
        pltpu.make_async_copy(keys_hbm.at[pl.ds(c * KCHUNK, KCHUNK)],
                              keybuf.at[par], sem_k).wait()

        @pl.when(c + 1 < NKCH)
        def _():
            pltpu.async_copy(keys_hbm.at[pl.ds((c + 1) * KCHUNK, KCHUNK)],
                             keybuf.at[(c + 1) % 2], sem_k)

        def vec_body(i4, acc):
            for u in range(4):
                i = i4 * 4 + u
                k16 = keybuf[par, pl.ds(i * 16, 16)]
                off = k16 - lo
                m = off.astype(jnp.uint32) < jnp.uint32(CPW)
                offs = jnp.where(m, off, 0)
                pvec = c * KCHUNK + i * 16 + iota16
                plsc.store_scatter(wid_l, [offs], pvec, mask=m)
                cur = plsc.load_gather(wid_l, [offs], mask=m)
                acc = acc | (m & (pvec > cur))
            return acc

        acc = lax.fori_loop(0, KCHUNK // 64, vec_body,
                            jnp.zeros((16,), jnp.bool_))

        @pl.when(jnp.any(acc))
        def _():
            def fix_vec(i, cc):
                k16 = keybuf[par, pl.ds(i * 16, 16)]
                off = k16 - lo
                m = off.astype(jnp.uint32) < jnp.uint32(CPW)
                offs = jnp.where(m, off, 0)
                pvec = c * KCHUNK + i * 16 + iota16
                cur = plsc.load_gather(wid_l, [offs], mask=m)
                need = m & (pvec > cur)

                def w_cond(st):
                    return st[0]

                def w_body(st):
                    nd = st[1]
                    plsc.store_scatter(wid_l, [offs], pvec, mask=nd)
                    cur2 = plsc.load_gather(wid_l, [offs], mask=m)
                    nd2 = m & (pvec > cur2)
                    return jnp.any(nd2), nd2

                lax.while_loop(w_cond, w_body, (jnp.any(need), need))
                return cc

            lax.fori_loop(0, KCHUNK // 16, fix_vec, 0)

        return carry

    lax.fori_loop(0, NKCH, chunk_body, 0)

    b_w = w // 8
    y0 = (w % 8) * NSEG

    def compress(sp, g2v, c3v):
        def comp_body(i, cnt):
            w16 = wid_l[pl.ds(sp * SEG + i * 16, 16)]
            m = w16 >= 0
            mi = m.astype(jnp.int32)
            pos = cnt + plsc.cumsum(mi) - 1
            poss = jnp.where(m, pos, 0)
            plsc.store_scatter(gidx, [g2v, poss], w16, mask=m)
            plsc.store_scatter(cpos, [c3v, poss], i * 16 + iota16, mask=m)
            return cnt + jnp.sum(mi)

        cnt = lax.fori_loop(0, SEG // 16, comp_body, jnp.int32(0))
        for t in range(WAVE // 16):
            plsc.store_scatter(gidx, [g2v, cnt + t * 16 + iota16],
                               t * 16 + iota16)
        return cnt

    def fire_wave(g2, t, rp):
        pltpu.async_copy(pf_hbm.at[gidx.at[g2, pl.ds(t * WAVE, WAVE)]],
                         rows.at[rp], sem_g)

    zeros16 = jnp.zeros((16,), jnp.int32)
    cnt0 = compress(jnp.int32(0), zeros16, zeros16)

    @pl.when(cnt0 > 0)
    def _():
        fire_wave(0, 0, 0)

    def piece_body(s, st):
        cnt_m2, cnt_m1, cnt, wp0 = st
        sp2 = s % 2
        sp3 = s % 3
        parv = zeros16 + sp2

        @pl.when(s >= 2)
        def _():
            pltpu.make_async_copy(stage.at[sp2],
                                  out_hbm.at[b_w, :, y0 + s, :],
                                  sem_o).wait()

            @functools.partial(plsc.parallel_loop, 0, (cnt_m2 + 15) // 16,
                               unroll=2)
            def rz_body(j):
                valid = (j * 16 + iota16) < cnt_m2
                cp16 = jnp.where(valid, cpos[(s + 1) % 3, pl.ds(j * 16, 16)],
                                 0)
                for f in range(F):
                    fv = jnp.full((16,), f, jnp.int32)
                    plsc.store_scatter(stage, [parv, fv, cp16],
                                       zero16f, mask=valid)

        spn = jnp.minimum(s + 1, NSEG - 1)
        cnt_nx = compress(spn, zeros16 + (s + 1) % 2, zeros16 + (s + 1) % 3)

        ntr = (cnt + WAVE - 1) // WAVE

        def wave_body(t, c):
            wp = (wp0 + t) % 2
            pltpu.make_async_copy(
                pf_hbm.at[gidx.at[sp2, pl.ds(t * WAVE, WAVE)]],
                rows.at[wp], sem_g).wait()

            @pl.when(t + 1 < ntr)
            def _():
                fire_wave(sp2, t + 1, (wp0 + t + 1) % 2)

            wpv = zeros16 + wp
            lc = cnt - t * WAVE
            jq = (jnp.minimum(lc, WAVE) + 15) // 16

            @functools.partial(plsc.parallel_loop, 0, jq, unroll=2)
            def q_body(q):
                valid = (q * 16 + iota16) < lc
                cp16 = jnp.where(
                    valid, cpos[sp3, pl.ds(t * WAVE + q * 16, 16)], 0)
                r16 = q * 16 + iota16
                for f in range(F):
                    fv = jnp.full((16,), f, jnp.int32)
                    vals = plsc.load_gather(rows, [wpv, r16, fv])
                    plsc.store_scatter(stage, [parv, fv, cp16], vals,
                                       mask=valid)

            return c

        lax.fori_loop(0, ntr, wave_body, 0)

        @pl.when((s + 1 < NSEG) & (cnt_nx > 0))
        def _():
            fire_wave((s + 1) % 2, 0, (wp0 + ntr) % 2)

        pltpu.async_copy(stage.at[sp2], out_hbm.at[b_w, :, y0 + s, :], sem_o)
        return (cnt_m1, cnt, cnt_nx, (wp0 + ntr) % 2)

    lax.fori_loop(0, NSEG, piece_body,
                  (jnp.int32(0), jnp.int32(0), cnt0, jnp.int32(0)))

    def drain_body(i, c):
        pltpu.make_async_copy(stage.at[0], out_hbm.at[0, :, 0, :],
                              sem_o).wait()
        return c

    lax.fori_loop(0, 2, drain_body, 0)


def kernel(pillar_features, voxel_coords, voxel_features):
    del voxel_features
    vc = voxel_coords.astype(jnp.int32)
    bcol = jnp.zeros((KP,), jnp.int32).at[:P].set(vc[:, 0])
    ycol = jnp.zeros((KP,), jnp.int32).at[:P].set(vc[:, 2])
    xcol = jnp.zeros((KP,), jnp.int32).at[:P].set(vc[:, 3])
    pf_pad = jnp.zeros((P + 16, FP), jnp.float32).at[:P, :F].set(pillar_features)
    keys = _keys_kernel(bcol, ycol, xcol)
    out = _scatter_kernel(keys, pf_pad)
    return out.reshape(B, F * NZ, NY, NX)
